# baseline (speedup 1.0000x reference)
import functools

import jax
import jax.numpy as jnp
from jax import lax
from jax.experimental import pallas as pl
from jax.experimental.pallas import tpu as pltpu
from jax.experimental.pallas import tpu_sc as plsc

VOCAB = 100000
D = 64
B = 1024
C = 20
NNEG = 20
PAD = 0

K = C + C * NNEG
KP = 432
OUTW = 512
NW = 32
BPW = B // NW
PD = 2 * D
PV = VOCAB // 2
CHUNKS = ((0, 128), (128, 128), (256, 128), (384, KP - 384))

_mesh = plsc.VectorSubcoreMesh(core_axis_name="c", subcore_axis_name="s")

_GDN = lax.GatherDimensionNumbers(
    offset_dims=(), collapsed_slice_dims=(0,), start_index_map=(0,))


def _take16(v, idx):
    return lax.gather(v, idx.reshape(16, 1), _GDN, (1,),
                      mode=lax.GatherScatterMode.PROMISE_IN_BOUNDS)


@functools.partial(
    pl.kernel,
    out_type=jax.ShapeDtypeStruct((B * OUTW,), jnp.float32),
    mesh=_mesh,
    scratch_types=[
        pltpu.VMEM((BPW + 16,), jnp.int32),
        pltpu.VMEM((BPW, PD), jnp.float32),
        pltpu.VMEM((KP,), jnp.int32),
        pltpu.VMEM((KP,), jnp.int32),
        pltpu.VMEM((KP,), jnp.int32),
        pltpu.VMEM((KP,), jnp.int32),
        pltpu.VMEM((KP,), jnp.int32),
        pltpu.VMEM((KP,), jnp.int32),
        pltpu.VMEM((128, PD), jnp.float32),
        pltpu.VMEM((128, PD), jnp.float32),
        pltpu.VMEM((OUTW,), jnp.float32),
        pltpu.VMEM((OUTW,), jnp.float32),
        pltpu.SemaphoreType.DMA,
        pltpu.SemaphoreType.DMA,
        pltpu.SemaphoreType.DMA,
        pltpu.SemaphoreType.DMA,
        pltpu.SemaphoreType.DMA,
    ],
    compiler_params=pltpu.CompilerParams(use_tc_tiling_on_sc=False,
                                         needs_layout_passes=False),
)
def _sc_scores(iword_hbm, okidx_hbm, opair_hbm, ipair_hbm, out_hbm,
               iw_v, iv_v, raw0, raw1, gix0, gix1, hof0, hof1,
               rowsA, rowsB, sc0, sc1, gsemA, gsemB, isem, ssem0, ssem1):
    wid = lax.axis_index("s") * 2 + lax.axis_index("c")
    base_b = wid * BPW
    pltpu.sync_copy(iword_hbm.at[pl.ds(base_b, BPW)], iw_v.at[pl.ds(0, BPW)])

    raw = (raw0, raw1)
    gix = (gix0, gix1)
    hof = (hof0, hof1)
    rowsbuf = (rowsA, rowsB)
    scb = (sc0, sc1)
    gsem = (gsemA, gsemB)
    ssem = (ssem0, ssem1)

    lane = lax.iota(jnp.int32, 16)
    brev = (((lane & 1) << 3) | ((lane & 2) << 1)
            | ((lane & 4) >> 1) | ((lane & 8) >> 3))
    folds = ((lane < 8, lane ^ 8), ((lane & 7) < 4, lane ^ 4),
             ((lane & 3) < 2, lane ^ 2), ((lane & 1) < 1, lane ^ 1))

    for q in range(BPW // 16):
        raw0[pl.ds(16 * q, 16)] = iw_v[pl.ds(16 * q, 16)] >> 1
    pltpu.async_copy(ipair_hbm.at[raw0.at[pl.ds(0, BPW)]], iv_v, gsemA)
    pltpu.make_async_copy(ipair_hbm.at[raw0.at[pl.ds(0, BPW)]], iv_v,
                          gsemA).wait()

    zeros16 = jnp.zeros((16,), jnp.float32)
    for sv in scb:
        for q in range((OUTW - KP) // 16):
            sv[pl.ds(KP + 16 * q, 16)] = zeros16

    def prep(qb):
        for q in range(KP // 16):
            v = raw[qb][pl.ds(16 * q, 16)]
            gix[qb][pl.ds(16 * q, 16)] = v >> 1
            hof[qb][pl.ds(16 * q, 16)] = v & 1

    def fire_chunk(qb, buf, ci):
        st, n = CHUNKS[ci]
        pltpu.async_copy(opair_hbm.at[gix[qb].at[pl.ds(st, n)]],
                         rowsbuf[buf].at[pl.ds(0, n)], gsem[buf])

    def wait_chunk(qb, buf, ci):
        st, n = CHUNKS[ci]
        pltpu.make_async_copy(opair_hbm.at[gix[qb].at[pl.ds(st, n)]],
                              rowsbuf[buf].at[pl.ds(0, n)],
                              gsem[buf]).wait()

    def compute_chunk(qb, buf, ci, bl):
        st, n = CHUNKS[ci]
        wv = iw_v[pl.ds(bl, 16)]
        wbit = _take16(wv, jnp.zeros_like(lane)) & 1
        wmask = wbit > 0
        iregs = [jnp.where(wmask,
                           iv_v[bl, pl.ds(D + 16 * q, 16)],
                           iv_v[bl, pl.ds(16 * q, 16)])
                 for q in range(4)]
        rv = rowsbuf[buf]
        sv = scb[qb]

        def g_body(g, carry2):
            row0 = g * 16
            lo = []
            hi = []
            for r in range(16):
                row = row0 + r
                lo.append(rv[row, pl.ds(0, 16)] * iregs[0]
                          + rv[row, pl.ds(16, 16)] * iregs[1]
                          + rv[row, pl.ds(32, 16)] * iregs[2]
                          + rv[row, pl.ds(48, 16)] * iregs[3])
                hi.append(rv[row, pl.ds(64, 16)] * iregs[0]
                          + rv[row, pl.ds(80, 16)] * iregs[1]
                          + rv[row, pl.ds(96, 16)] * iregs[2]
                          + rv[row, pl.ds(112, 16)] * iregs[3])
            for m, rt in folds:
                lo = [jnp.where(m, a + _take16(a, rt), b2 + _take16(b2, rt))
                      for a, b2 in zip(lo[::2], lo[1::2])]
                hi = [jnp.where(m, a + _take16(a, rt), b2 + _take16(b2, rt))
                      for a, b2 in zip(hi[::2], hi[1::2])]
            hv = hof[qb][pl.ds(st + row0, 16)]
            hvb = _take16(hv, brev)
            res = jnp.where(hvb > 0, hi[0], lo[0])
            plsc.store_scatter(sv, [(st + row0) + brev], res)
            return carry2

        lax.fori_loop(0, n // 16, g_body, 0)

    def writeback(qb, b):
        for t in range(4):
            pltpu.async_copy(scb[qb].at[pl.ds(t * 128, 128)],
                             out_hbm.at[pl.ds(b * OUTW + t * 128, 128)],
                             ssem[qb])

    def wait_writeback(qb, b_old):
        for t in range(4):
            pltpu.make_async_copy(scb[qb].at[pl.ds(t * 128, 128)],
                                  out_hbm.at[pl.ds(b_old * OUTW + t * 128,
                                                   128)],
                                  ssem[qb]).wait()

    pltpu.sync_copy(okidx_hbm.at[base_b], raw[0])
    prep(0)
    pltpu.sync_copy(okidx_hbm.at[base_b + 1], raw[1])
    fire_chunk(0, 0, 0)

    def pair_body(i, carry):
        for par in range(2):
            bl = 2 * i + par
            b = base_b + bl
            qb = par
            qn = 1 - par
            for ci in range(4):
                buf = ci & 1
                if ci < 3:
                    fire_chunk(qb, 1 - buf, ci + 1)
                else:
                    @pl.when(bl + 1 < BPW)
                    def _():
                        fire_chunk(qn, 1 - buf, 0)
                if ci == 1:
                    @pl.when(bl + 1 < BPW)
                    def _():
                        @pl.when(bl >= 1)
                        def _():
                            pltpu.make_async_copy(okidx_hbm.at[b + 1],
                                                  raw[qn], isem).wait()
                        prep(qn)

                        @pl.when(bl + 2 < BPW)
                        def _():
                            pltpu.async_copy(okidx_hbm.at[b + 2], raw[qb],
                                             isem)
                if ci == 0:
                    @pl.when(bl >= 2)
                    def _():
                        wait_writeback(qb, b - 2)
                wait_chunk(qb, buf, ci)
                compute_chunk(qb, buf, ci, bl)
            writeback(qb, b)
        return carry

    lax.fori_loop(0, BPW // 2, pair_body, 0)
    wait_writeback(0, base_b + BPW - 2)
    wait_writeback(1, base_b + BPW - 1)


def _tc_loss_body(s_ref, ow_ref, out_ref):
    s = s_ref[...]
    ow4 = ow_ref[...]

    def log_sigmoid(x):
        return jnp.minimum(x, 0.0) - jnp.log1p(jnp.exp(-jnp.abs(x)))

    rr = lax.broadcasted_iota(jnp.int32, s.shape, 0)
    ll = lax.broadcasted_iota(jnp.int32, s.shape, 1)
    col = (rr & 3) * 128 + ll
    omask = (col < C) & (ow4 != PAD)
    nmask = (col >= C) & (col < K)
    omaskf = omask.astype(jnp.float32)
    nmaskf = nmask.astype(jnp.float32)
    n_valid = jnp.sum(omaskf)
    oloss = jnp.sum(log_sigmoid(s) * omaskf) / n_valid
    nterm = jnp.sum(log_sigmoid(-s) * nmaskf) / (C * B)
    out_ref[0, 0] = -(oloss + nterm)


def _tc_loss(scores4, ow4):
    return pl.pallas_call(
        _tc_loss_body,
        out_shape=jax.ShapeDtypeStruct((1, 1), jnp.float32),
        in_specs=[
            pl.BlockSpec(memory_space=pltpu.VMEM),
            pl.BlockSpec(memory_space=pltpu.VMEM),
        ],
        out_specs=pl.BlockSpec(memory_space=pltpu.SMEM),
    )(scores4, ow4)


def kernel(iword, owords, nwords, ivec_table, ovec_table):
    pad = jnp.zeros((B, KP - K), jnp.int32)
    okidx = jnp.concatenate([owords, nwords, pad], axis=1)
    opair = ovec_table.reshape(PV, PD)
    ipair = ivec_table.reshape(PV, PD)
    flat = _sc_scores(iword, okidx, opair, ipair)
    scores4 = flat.reshape(B * 4, 128)
    ow128 = jnp.pad(owords, ((0, 0), (0, 128 - C)))
    ow4 = jnp.concatenate(
        [ow128[:, None, :], jnp.ones((B, 3, 128), jnp.int32)],
        axis=1).reshape(B * 4, 128)
    loss = _tc_loss(scores4, ow4)
    return loss[0, 0]


# speedup vs baseline: 1.8197x; 1.8197x over previous
import functools

import jax
import jax.numpy as jnp
from jax import lax
from jax.experimental import pallas as pl
from jax.experimental.pallas import tpu as pltpu
from jax.experimental.pallas import tpu_sc as plsc

VOCAB = 100000
D = 64
B = 1024
C = 20
NNEG = 20
PAD = 0

K = C + C * NNEG
KP = 432
OUTW = 512
NW = 32
BPW = B // NW
PD = 2 * D
PV = VOCAB // 2
HALVES = ((0, 208, ((0, 128), (128, 80))),
          (208, 224, ((208, 128), (336, 88))))
HBUF = 224

_mesh = plsc.VectorSubcoreMesh(core_axis_name="c", subcore_axis_name="s")

_GDN = lax.GatherDimensionNumbers(
    offset_dims=(), collapsed_slice_dims=(0,), start_index_map=(0,))


def _take16(v, idx):
    return lax.gather(v, idx.reshape(16, 1), _GDN, (1,),
                      mode=lax.GatherScatterMode.PROMISE_IN_BOUNDS)


@functools.partial(
    pl.kernel,
    out_type=jax.ShapeDtypeStruct((B * OUTW,), jnp.float32),
    mesh=_mesh,
    scratch_types=[
        pltpu.VMEM((BPW + 16,), jnp.int32),
        pltpu.VMEM((BPW, PD), jnp.float32),
        pltpu.VMEM((KP,), jnp.int32),
        pltpu.VMEM((KP,), jnp.int32),
        pltpu.VMEM((KP,), jnp.int32),
        pltpu.VMEM((KP,), jnp.int32),
        pltpu.VMEM((KP,), jnp.int32),
        pltpu.VMEM((KP,), jnp.int32),
        pltpu.VMEM((HBUF, PD), jnp.float32),
        pltpu.VMEM((HBUF, PD), jnp.float32),
        pltpu.VMEM((OUTW,), jnp.float32),
        pltpu.VMEM((OUTW,), jnp.float32),
        pltpu.SemaphoreType.DMA,
        pltpu.SemaphoreType.DMA,
        pltpu.SemaphoreType.DMA,
        pltpu.SemaphoreType.DMA,
        pltpu.SemaphoreType.DMA,
    ],
    compiler_params=pltpu.CompilerParams(use_tc_tiling_on_sc=False,
                                         needs_layout_passes=False),
)
def _sc_scores(iword_hbm, okidx_hbm, opair_hbm, ipair_hbm, out_hbm,
               iw_v, iv_v, raw0, raw1, gix0, gix1, hof0, hof1,
               rowsA, rowsB, sc0, sc1, gsemA, gsemB, isem, ssem0, ssem1):
    wid = lax.axis_index("s") * 2 + lax.axis_index("c")
    base_b = wid * BPW
    pltpu.sync_copy(iword_hbm.at[pl.ds(base_b, BPW)], iw_v.at[pl.ds(0, BPW)])

    raw = (raw0, raw1)
    gix = (gix0, gix1)
    hof = (hof0, hof1)
    rowsbuf = (rowsA, rowsB)
    scb = (sc0, sc1)
    gsem = (gsemA, gsemB)
    ssem = (ssem0, ssem1)

    lane = lax.iota(jnp.int32, 16)
    brev = (((lane & 1) << 3) | ((lane & 2) << 1)
            | ((lane & 4) >> 1) | ((lane & 8) >> 3))
    folds = ((lane < 8, lane ^ 8), ((lane & 7) < 4, lane ^ 4),
             ((lane & 3) < 2, lane ^ 2), ((lane & 1) < 1, lane ^ 1))

    for q in range(BPW // 16):
        raw0[pl.ds(16 * q, 16)] = iw_v[pl.ds(16 * q, 16)] >> 1
    pltpu.async_copy(ipair_hbm.at[raw0.at[pl.ds(0, BPW)]], iv_v, gsemA)
    pltpu.make_async_copy(ipair_hbm.at[raw0.at[pl.ds(0, BPW)]], iv_v,
                          gsemA).wait()

    zeros16 = jnp.zeros((16,), jnp.float32)
    for sv in scb:
        for q in range((OUTW - KP) // 16):
            sv[pl.ds(KP + 16 * q, 16)] = zeros16

    def prep(qb):
        for q in range(KP // 16):
            v = raw[qb][pl.ds(16 * q, 16)]
            gix[qb][pl.ds(16 * q, 16)] = v >> 1
            hof[qb][pl.ds(16 * q, 16)] = v & 1

    def fire_half(qb, buf, h):
        hst, hn, chunks = HALVES[h]
        for st, n in chunks:
            pltpu.async_copy(opair_hbm.at[gix[qb].at[pl.ds(st, n)]],
                             rowsbuf[buf].at[pl.ds(st - hst, n)], gsem[buf])

    def wait_half(qb, buf, h):
        hst, hn, chunks = HALVES[h]
        for st, n in chunks:
            pltpu.make_async_copy(opair_hbm.at[gix[qb].at[pl.ds(st, n)]],
                                  rowsbuf[buf].at[pl.ds(st - hst, n)],
                                  gsem[buf]).wait()

    def compute_half(qb, buf, h, bl):
        hst, hn, _chunks = HALVES[h]
        wv = iw_v[pl.ds(bl, 16)]
        wbit = _take16(wv, jnp.zeros_like(lane)) & 1
        wmask = wbit > 0
        iregs = [jnp.where(wmask,
                           iv_v[bl, pl.ds(D + 16 * q, 16)],
                           iv_v[bl, pl.ds(16 * q, 16)])
                 for q in range(4)]
        rv = rowsbuf[buf]
        sv = scb[qb]

        def g_body(g, carry2):
            row0 = g * 16
            lo = []
            hi = []
            for r in range(16):
                row = row0 + r
                lo.append(rv[row, pl.ds(0, 16)] * iregs[0]
                          + rv[row, pl.ds(16, 16)] * iregs[1]
                          + rv[row, pl.ds(32, 16)] * iregs[2]
                          + rv[row, pl.ds(48, 16)] * iregs[3])
                hi.append(rv[row, pl.ds(64, 16)] * iregs[0]
                          + rv[row, pl.ds(80, 16)] * iregs[1]
                          + rv[row, pl.ds(96, 16)] * iregs[2]
                          + rv[row, pl.ds(112, 16)] * iregs[3])
            for m, rt in folds:
                lo = [jnp.where(m, a + _take16(a, rt), b2 + _take16(b2, rt))
                      for a, b2 in zip(lo[::2], lo[1::2])]
                hi = [jnp.where(m, a + _take16(a, rt), b2 + _take16(b2, rt))
                      for a, b2 in zip(hi[::2], hi[1::2])]
            hv = hof[qb][pl.ds(hst + row0, 16)]
            hvb = _take16(hv, brev)
            res = jnp.where(hvb > 0, hi[0], lo[0])
            plsc.store_scatter(sv, [(hst + row0) + brev], res)
            return carry2

        lax.fori_loop(0, hn // 16, g_body, 0)

    def writeback(qb, b):
        for t in range(4):
            pltpu.async_copy(scb[qb].at[pl.ds(t * 128, 128)],
                             out_hbm.at[pl.ds(b * OUTW + t * 128, 128)],
                             ssem[qb])

    def wait_writeback(qb, b_old):
        for t in range(4):
            pltpu.make_async_copy(scb[qb].at[pl.ds(t * 128, 128)],
                                  out_hbm.at[pl.ds(b_old * OUTW + t * 128,
                                                   128)],
                                  ssem[qb]).wait()

    pltpu.sync_copy(okidx_hbm.at[base_b], raw[0])
    prep(0)
    pltpu.sync_copy(okidx_hbm.at[base_b + 1], raw[1])
    fire_half(0, 0, 0)

    def pair_body(i, carry):
        for par in range(2):
            bl = 2 * i + par
            b = base_b + bl
            qb = par
            qn = 1 - par
            for h in range(2):
                if h == 0:
                    fire_half(qb, 1, 1)
                    @pl.when(bl + 1 < BPW)
                    def _():
                        @pl.when(bl >= 1)
                        def _():
                            pltpu.make_async_copy(okidx_hbm.at[b + 1],
                                                  raw[qn], isem).wait()
                        prep(qn)

                        @pl.when(bl + 2 < BPW)
                        def _():
                            pltpu.async_copy(okidx_hbm.at[b + 2], raw[qb],
                                             isem)
                    @pl.when(bl >= 2)
                    def _():
                        wait_writeback(qb, b - 2)
                else:
                    @pl.when(bl + 1 < BPW)
                    def _():
                        fire_half(qn, 0, 0)
                wait_half(qb, h, h)
                compute_half(qb, h, h, bl)
            writeback(qb, b)
        return carry

    lax.fori_loop(0, BPW // 2, pair_body, 0)
    wait_writeback(0, base_b + BPW - 2)
    wait_writeback(1, base_b + BPW - 1)


def _tc_loss_body(s_ref, ow_ref, out_ref):
    s = s_ref[...]
    ow4 = ow_ref[...]

    def log_sigmoid(x):
        return jnp.minimum(x, 0.0) - jnp.log1p(jnp.exp(-jnp.abs(x)))

    rr = lax.broadcasted_iota(jnp.int32, s.shape, 0)
    ll = lax.broadcasted_iota(jnp.int32, s.shape, 1)
    col = (rr & 3) * 128 + ll
    omask = (col < C) & (ow4 != PAD)
    nmask = (col >= C) & (col < K)
    omaskf = omask.astype(jnp.float32)
    nmaskf = nmask.astype(jnp.float32)
    n_valid = jnp.sum(omaskf)
    oloss = jnp.sum(log_sigmoid(s) * omaskf) / n_valid
    nterm = jnp.sum(log_sigmoid(-s) * nmaskf) / (C * B)
    out_ref[0, 0] = -(oloss + nterm)


def _tc_loss(scores4, ow4):
    return pl.pallas_call(
        _tc_loss_body,
        out_shape=jax.ShapeDtypeStruct((1, 1), jnp.float32),
        in_specs=[
            pl.BlockSpec(memory_space=pltpu.VMEM),
            pl.BlockSpec(memory_space=pltpu.VMEM),
        ],
        out_specs=pl.BlockSpec(memory_space=pltpu.SMEM),
    )(scores4, ow4)


def kernel(iword, owords, nwords, ivec_table, ovec_table):
    pad = jnp.zeros((B, KP - K), jnp.int32)
    okidx = jnp.concatenate([owords, nwords, pad], axis=1)
    opair = ovec_table.reshape(PV, PD)
    ipair = ivec_table.reshape(PV, PD)
    flat = _sc_scores(iword, okidx, opair, ipair)
    scores4 = flat.reshape(B * 4, 128)
    ow128 = jnp.pad(owords, ((0, 0), (0, 128 - C)))
    ow4 = jnp.concatenate(
        [ow128[:, None, :], jnp.ones((B, 3, 128), jnp.int32)],
        axis=1).reshape(B * 4, 128)
    loss = _tc_loss(scores4, ow4)
    return loss[0, 0]


# speedup vs baseline: 1.8232x; 1.0019x over previous
import functools

import jax
import jax.numpy as jnp
from jax import lax
from jax.experimental import pallas as pl
from jax.experimental.pallas import tpu as pltpu
from jax.experimental.pallas import tpu_sc as plsc

VOCAB = 100000
D = 64
B = 1024
C = 20
NNEG = 20
PAD = 0

K = C + C * NNEG
KP = 432
OUTW = 512
NW = 32
BPW = B // NW
PD = 2 * D
PV = VOCAB // 2
HALVES = ((0, 208, ((0, 128), (128, 80))),
          (208, 224, ((208, 128), (336, 88))))
HBUF = 224

_mesh = plsc.VectorSubcoreMesh(core_axis_name="c", subcore_axis_name="s")

_GDN = lax.GatherDimensionNumbers(
    offset_dims=(), collapsed_slice_dims=(0,), start_index_map=(0,))


def _take16(v, idx):
    return lax.gather(v, idx.reshape(16, 1), _GDN, (1,),
                      mode=lax.GatherScatterMode.PROMISE_IN_BOUNDS)


@functools.partial(
    pl.kernel,
    out_type=jax.ShapeDtypeStruct((B * OUTW,), jnp.float32),
    mesh=_mesh,
    scratch_types=[
        pltpu.VMEM((BPW + 16,), jnp.int32),
        pltpu.VMEM((BPW, PD), jnp.float32),
        pltpu.VMEM((KP,), jnp.int32),
        pltpu.VMEM((KP,), jnp.int32),
        pltpu.VMEM((KP,), jnp.int32),
        pltpu.VMEM((KP,), jnp.int32),
        pltpu.VMEM((KP,), jnp.int32),
        pltpu.VMEM((KP,), jnp.int32),
        pltpu.VMEM((HBUF, PD), jnp.float32),
        pltpu.VMEM((HBUF, PD), jnp.float32),
        pltpu.VMEM((OUTW,), jnp.float32),
        pltpu.VMEM((OUTW,), jnp.float32),
        pltpu.SemaphoreType.DMA,
        pltpu.SemaphoreType.DMA,
        pltpu.SemaphoreType.DMA,
        pltpu.SemaphoreType.DMA,
        pltpu.SemaphoreType.DMA,
    ],
    compiler_params=pltpu.CompilerParams(use_tc_tiling_on_sc=False,
                                         needs_layout_passes=False),
)
def _sc_scores(iword_hbm, okidx_hbm, opair_hbm, ipair_hbm, out_hbm,
               iw_v, iv_v, raw0, raw1, gix0, gix1, hof0, hof1,
               rowsA, rowsB, sc0, sc1, gsemA, gsemB, isem, ssem0, ssem1):
    wid = lax.axis_index("s") * 2 + lax.axis_index("c")
    base_b = wid * BPW
    pltpu.sync_copy(iword_hbm.at[pl.ds(base_b, BPW)], iw_v.at[pl.ds(0, BPW)])

    raw = (raw0, raw1)
    gix = (gix0, gix1)
    hof = (hof0, hof1)
    rowsbuf = (rowsA, rowsB)
    scb = (sc0, sc1)
    gsem = (gsemA, gsemB)
    ssem = (ssem0, ssem1)

    lane = lax.iota(jnp.int32, 16)
    brev = (((lane & 1) << 3) | ((lane & 2) << 1)
            | ((lane & 4) >> 1) | ((lane & 8) >> 3))
    folds = ((lane < 8, lane ^ 8), ((lane & 7) < 4, lane ^ 4),
             ((lane & 3) < 2, lane ^ 2), ((lane & 1) < 1, lane ^ 1))

    for q in range(BPW // 16):
        raw0[pl.ds(16 * q, 16)] = iw_v[pl.ds(16 * q, 16)] >> 1
    pltpu.async_copy(ipair_hbm.at[raw0.at[pl.ds(0, BPW)]], iv_v, gsemA)
    pltpu.make_async_copy(ipair_hbm.at[raw0.at[pl.ds(0, BPW)]], iv_v,
                          gsemA).wait()

    zeros16 = jnp.zeros((16,), jnp.float32)
    for sv in scb:
        for q in range((OUTW - KP) // 16):
            sv[pl.ds(KP + 16 * q, 16)] = zeros16

    def prep(qb):
        for q in range(KP // 16):
            v = raw[qb][pl.ds(16 * q, 16)]
            gix[qb][pl.ds(16 * q, 16)] = v >> 1
            hof[qb][pl.ds(16 * q, 16)] = v & 1

    def fire_half(qb, buf, h):
        hst, hn, chunks = HALVES[h]
        for st, n in chunks:
            pltpu.async_copy(opair_hbm.at[gix[qb].at[pl.ds(st, n)]],
                             rowsbuf[buf].at[pl.ds(st - hst, n)], gsem[buf])

    def wait_half(qb, buf, h):
        hst, hn, chunks = HALVES[h]
        for st, n in chunks:
            pltpu.make_async_copy(opair_hbm.at[gix[qb].at[pl.ds(st, n)]],
                                  rowsbuf[buf].at[pl.ds(st - hst, n)],
                                  gsem[buf]).wait()

    def compute_half(qb, buf, h, bl):
        hst, hn, _chunks = HALVES[h]
        wv = iw_v[pl.ds(bl, 16)]
        wbit = _take16(wv, jnp.zeros_like(lane)) & 1
        wmask = wbit > 0
        iregs = [jnp.where(wmask,
                           iv_v[bl, pl.ds(D + 16 * q, 16)],
                           iv_v[bl, pl.ds(16 * q, 16)])
                 for q in range(4)]
        rv = rowsbuf[buf]
        sv = scb[qb]

        def g_body(g, carry2):
            row0 = g * 16
            hv = hof[qb][pl.ds(hst + row0, 16)]
            cur = []
            for r in range(16):
                row = row0 + r
                lo = (rv[row, pl.ds(0, 16)] * iregs[0]
                      + rv[row, pl.ds(16, 16)] * iregs[1]
                      + rv[row, pl.ds(32, 16)] * iregs[2]
                      + rv[row, pl.ds(48, 16)] * iregs[3])
                hi = (rv[row, pl.ds(64, 16)] * iregs[0]
                      + rv[row, pl.ds(80, 16)] * iregs[1]
                      + rv[row, pl.ds(96, 16)] * iregs[2]
                      + rv[row, pl.ds(112, 16)] * iregs[3])
                hsel = _take16(hv, lane & 0 | r) > 0
                cur.append(jnp.where(hsel, hi, lo))
            for m, rt in folds:
                cur = [jnp.where(m, a + _take16(a, rt), b2 + _take16(b2, rt))
                       for a, b2 in zip(cur[::2], cur[1::2])]
            plsc.store_scatter(sv, [(hst + row0) + brev], cur[0])
            return carry2

        lax.fori_loop(0, hn // 16, g_body, 0)

    def writeback(qb, b):
        pltpu.async_copy(scb[qb], out_hbm.at[pl.ds(b * OUTW, OUTW)],
                         ssem[qb])

    def wait_writeback(qb, b_old):
        pltpu.make_async_copy(scb[qb], out_hbm.at[pl.ds(b_old * OUTW, OUTW)],
                              ssem[qb]).wait()

    pltpu.sync_copy(okidx_hbm.at[base_b], raw[0])
    prep(0)
    pltpu.sync_copy(okidx_hbm.at[base_b + 1], raw[1])
    fire_half(0, 0, 0)

    def pair_body(i, carry):
        for par in range(2):
            bl = 2 * i + par
            b = base_b + bl
            qb = par
            qn = 1 - par
            for h in range(2):
                if h == 0:
                    fire_half(qb, 1, 1)
                    @pl.when(bl + 1 < BPW)
                    def _():
                        @pl.when(bl >= 1)
                        def _():
                            pltpu.make_async_copy(okidx_hbm.at[b + 1],
                                                  raw[qn], isem).wait()
                        prep(qn)

                        @pl.when(bl + 2 < BPW)
                        def _():
                            pltpu.async_copy(okidx_hbm.at[b + 2], raw[qb],
                                             isem)
                    @pl.when(bl >= 2)
                    def _():
                        wait_writeback(qb, b - 2)
                else:
                    @pl.when(bl + 1 < BPW)
                    def _():
                        fire_half(qn, 0, 0)
                wait_half(qb, h, h)
                compute_half(qb, h, h, bl)
            writeback(qb, b)
        return carry

    lax.fori_loop(0, BPW // 2, pair_body, 0)
    wait_writeback(0, base_b + BPW - 2)
    wait_writeback(1, base_b + BPW - 1)


def _tc_loss_body(s_ref, ow_ref, out_ref):
    s = s_ref[...]
    ow4 = ow_ref[...]

    def log_sigmoid(x):
        return jnp.minimum(x, 0.0) - jnp.log1p(jnp.exp(-jnp.abs(x)))

    rr = lax.broadcasted_iota(jnp.int32, s.shape, 0)
    ll = lax.broadcasted_iota(jnp.int32, s.shape, 1)
    col = (rr & 3) * 128 + ll
    omask = (col < C) & (ow4 != PAD)
    nmask = (col >= C) & (col < K)
    omaskf = omask.astype(jnp.float32)
    nmaskf = nmask.astype(jnp.float32)
    n_valid = jnp.sum(omaskf)
    oloss = jnp.sum(log_sigmoid(s) * omaskf) / n_valid
    nterm = jnp.sum(log_sigmoid(-s) * nmaskf) / (C * B)
    out_ref[0, 0] = -(oloss + nterm)


def _tc_loss(scores4, ow4):
    return pl.pallas_call(
        _tc_loss_body,
        out_shape=jax.ShapeDtypeStruct((1, 1), jnp.float32),
        in_specs=[
            pl.BlockSpec(memory_space=pltpu.VMEM),
            pl.BlockSpec(memory_space=pltpu.VMEM),
        ],
        out_specs=pl.BlockSpec(memory_space=pltpu.SMEM),
    )(scores4, ow4)


def kernel(iword, owords, nwords, ivec_table, ovec_table):
    pad = jnp.zeros((B, KP - K), jnp.int32)
    okidx = jnp.concatenate([owords, nwords, pad], axis=1)
    opair = ovec_table.reshape(PV, PD)
    ipair = ivec_table.reshape(PV, PD)
    flat = _sc_scores(iword, okidx, opair, ipair)
    scores4 = flat.reshape(B * 4, 128)
    ow128 = jnp.pad(owords, ((0, 0), (0, 128 - C)))
    ow4 = jnp.concatenate(
        [ow128[:, None, :], jnp.ones((B, 3, 128), jnp.int32)],
        axis=1).reshape(B * 4, 128)
    loss = _tc_loss(scores4, ow4)
    return loss[0, 0]
